# baseline (speedup 1.0000x reference)
import functools

import jax
import jax.numpy as jnp
from jax import lax
from jax.experimental import pallas as pl
from jax.experimental.pallas import tpu as pltpu
from jax.experimental.pallas import tpu_sc as plsc

ITEMS = 100000
EMBED = 32
B = 8
SEG = 1024
NPOS = B * SEG

NC = 2
NS = 16
NW = NC * NS
PER_W = NPOS // NW
CHUNK = 128
NCH = PER_W // CHUNK

_SC_MESH = plsc.VectorSubcoreMesh(
    core_axis_name="c", subcore_axis_name="s", num_cores=NC, num_subcores=NS)
_SC_PARAMS = pltpu.CompilerParams(use_tc_tiling_on_sc=False)


@functools.partial(
    pl.kernel,
    out_type=(
        jax.ShapeDtypeStruct((NPOS, EMBED), jnp.float32),
        jax.ShapeDtypeStruct((NPOS, 16), jnp.float32),
    ),
    mesh=_SC_MESH,
    scratch_types=(
        pltpu.VMEM((NCH, CHUNK), jnp.int32),
        pltpu.VMEM((PER_W, EMBED), jnp.float32),
        pltpu.VMEM((PER_W, 16), jnp.float32),
        pltpu.SemaphoreType.DMA,
    ),
    compiler_params=_SC_PARAMS,
)
def _sc_gather(nodes2_hbm, table_hbm, alpha_hbm, trow_hbm, aval_hbm,
               idx_v, rows_v, a_v, sem):
    wid = lax.axis_index("s") * NC + lax.axis_index("c")
    base = wid * PER_W
    pltpu.sync_copy(nodes2_hbm.at[pl.ds(wid * NCH, NCH)], idx_v)
    copies = []
    for j in range(NCH):
        copies.append(pltpu.async_copy(
            table_hbm.at[idx_v.at[j]], rows_v.at[pl.ds(j * CHUNK, CHUNK)], sem))
        copies.append(pltpu.async_copy(
            alpha_hbm.at[idx_v.at[j]], a_v.at[pl.ds(j * CHUNK, CHUNK)], sem))
    for c in copies:
        c.wait()
    pltpu.sync_copy(rows_v, trow_hbm.at[pl.ds(base, PER_W)])
    pltpu.sync_copy(a_v, aval_hbm.at[pl.ds(base, PER_W)])


def _val_body(nodes_ref, feat_ref, trow_ref, a_ref, val_ref):
    seg = nodes_ref[0, 0, :]
    eq = (seg[:, None] == seg[None, :]).astype(jnp.float32)
    counts = jnp.sum(eq, axis=1)
    sums = jnp.dot(eq, feat_ref[0],
                   preferred_element_type=jnp.float32,
                   precision=lax.Precision.HIGHEST)
    mean = sums / counts[:, None]
    a = a_ref[0, 0, :][:, None]
    val_ref[0] = (1.0 - a) * trow_ref[0] + a * mean


def _tc_val(nodes3, feat3, trow3, a3):
    return pl.pallas_call(
        _val_body,
        grid=(B,),
        in_specs=[
            pl.BlockSpec((1, 1, SEG), lambda g: (g, 0, 0)),
            pl.BlockSpec((1, SEG, EMBED), lambda g: (g, 0, 0)),
            pl.BlockSpec((1, SEG, EMBED), lambda g: (g, 0, 0)),
            pl.BlockSpec((1, 1, SEG), lambda g: (g, 0, 0)),
        ],
        out_specs=pl.BlockSpec((1, SEG, EMBED), lambda g: (g, 0, 0)),
        out_shape=jax.ShapeDtypeStruct((B, SEG, EMBED), jnp.float32),
    )(nodes3, feat3, trow3, a3)


BLK = 25000
RB = ITEMS // BLK


def _prefill_body(table_ref, out_ref):
    out_ref[...] = table_ref[...]


def _tc_prefill(table):
    return pl.pallas_call(
        _prefill_body,
        grid=(RB, B),
        in_specs=[pl.BlockSpec((BLK, EMBED), lambda rb, g: (rb, 0))],
        out_specs=pl.BlockSpec((BLK, EMBED), lambda rb, g: (g * RB + rb, 0)),
        out_shape=jax.ShapeDtypeStruct((B * ITEMS, EMBED), jnp.float32),
    )(table)


@functools.partial(
    pl.kernel,
    out_type=(),
    mesh=_SC_MESH,
    scratch_types=(
        pltpu.VMEM((NCH, CHUNK), jnp.int32),
        pltpu.VMEM((PER_W, EMBED), jnp.float32),
        pltpu.SemaphoreType.DMA,
    ),
    compiler_params=_SC_PARAMS,
)
def _sc_scatter(out_ref, nodes2_hbm, val_hbm, idx_v, val_v, sem):
    wid = lax.axis_index("s") * NC + lax.axis_index("c")
    base = wid * PER_W
    g = base // SEG
    pltpu.sync_copy(nodes2_hbm.at[pl.ds(wid * NCH, NCH)], idx_v)
    pltpu.sync_copy(val_hbm.at[pl.ds(base, PER_W)], val_v)
    off = g * ITEMS
    for j in range(NCH):
        for k in range(CHUNK // 16):
            sl = pl.ds(k * 16, 16)
            idx_v[j, sl] = idx_v[j, sl] + off
    copies = []
    for j in range(NCH):
        copies.append(pltpu.async_copy(
            val_v.at[pl.ds(j * CHUNK, CHUNK)], out_ref.at[idx_v.at[j]], sem))
    for c in copies:
        c.wait()


def kernel(nodes, nodes_output, ptr, table, alpha):
    del ptr
    nodes2 = nodes.reshape(NPOS // CHUNK, CHUNK)
    alpha16 = jnp.broadcast_to(alpha, (ITEMS, 16))
    trow, aval16 = _sc_gather(nodes2, table, alpha16)
    aval = aval16[:, :1]

    nodes3 = nodes.reshape(B, 1, SEG)
    feat3 = nodes_output.reshape(B, SEG, EMBED)
    trow3 = trow.reshape(B, SEG, EMBED)
    a3 = aval.reshape(B, 1, SEG)
    val = _tc_val(nodes3, feat3, trow3, a3)

    prefilled = _tc_prefill(table)
    out_ref = jax.new_ref(prefilled)
    _sc_scatter(out_ref, nodes2, val.reshape(NPOS, EMBED))
    return out_ref[...].reshape(B, ITEMS, EMBED)


# speedup vs baseline: 1.5439x; 1.5439x over previous
import functools

import jax
import jax.numpy as jnp
from jax import lax
from jax.experimental import pallas as pl
from jax.experimental.pallas import tpu as pltpu
from jax.experimental.pallas import tpu_sc as plsc

ITEMS = 100000
EMBED = 32
B = 8
SEG = 1024
NPOS = B * SEG

NC = 2
NS = 16
NW = NC * NS
PER_W = NPOS // NW
CHUNK = 128
NCH = PER_W // CHUNK

_SC_MESH = plsc.VectorSubcoreMesh(
    core_axis_name="c", subcore_axis_name="s", num_cores=NC, num_subcores=NS)
_SC_PARAMS = pltpu.CompilerParams(use_tc_tiling_on_sc=False)


@functools.partial(
    pl.kernel,
    out_type=(
        jax.ShapeDtypeStruct((NPOS, EMBED), jnp.float32),
        jax.ShapeDtypeStruct((NPOS, 16), jnp.float32),
    ),
    mesh=_SC_MESH,
    scratch_types=(
        pltpu.VMEM((NCH, CHUNK), jnp.int32),
        pltpu.VMEM((PER_W, EMBED), jnp.float32),
        pltpu.VMEM((PER_W, 16), jnp.float32),
        pltpu.SemaphoreType.DMA,
    ),
    compiler_params=_SC_PARAMS,
)
def _sc_gather(nodes2_hbm, table_hbm, alpha_hbm, trow_hbm, aval_hbm,
               idx_v, rows_v, a_v, sem):
    wid = lax.axis_index("s") * NC + lax.axis_index("c")
    base = wid * PER_W
    pltpu.sync_copy(nodes2_hbm.at[pl.ds(wid * NCH, NCH)], idx_v)
    copies = []
    for j in range(NCH):
        copies.append(pltpu.async_copy(
            table_hbm.at[idx_v.at[j]], rows_v.at[pl.ds(j * CHUNK, CHUNK)], sem))
        copies.append(pltpu.async_copy(
            alpha_hbm.at[idx_v.at[j]], a_v.at[pl.ds(j * CHUNK, CHUNK)], sem))
    for c in copies:
        c.wait()
    pltpu.sync_copy(rows_v, trow_hbm.at[pl.ds(base, PER_W)])
    pltpu.sync_copy(a_v, aval_hbm.at[pl.ds(base, PER_W)])


def _val_body(nodes_ref, feat_ref, trow_ref, a_ref, val_ref):
    seg = nodes_ref[0, 0, :]
    eq = (seg[:, None] == seg[None, :]).astype(jnp.float32)
    counts = jnp.sum(eq, axis=1)
    sums = jnp.dot(eq, feat_ref[0],
                   preferred_element_type=jnp.float32,
                   precision=lax.Precision.HIGHEST)
    mean = sums / counts[:, None]
    a = a_ref[0, 0, :][:, None]
    val_ref[0] = (1.0 - a) * trow_ref[0] + a * mean


def _tc_val(nodes3, feat3, trow3, a3):
    return pl.pallas_call(
        _val_body,
        grid=(B,),
        in_specs=[
            pl.BlockSpec((1, 1, SEG), lambda g: (g, 0, 0)),
            pl.BlockSpec((1, SEG, EMBED), lambda g: (g, 0, 0)),
            pl.BlockSpec((1, SEG, EMBED), lambda g: (g, 0, 0)),
            pl.BlockSpec((1, 1, SEG), lambda g: (g, 0, 0)),
        ],
        out_specs=pl.BlockSpec((1, SEG, EMBED), lambda g: (g, 0, 0)),
        out_shape=jax.ShapeDtypeStruct((B, SEG, EMBED), jnp.float32),
    )(nodes3, feat3, trow3, a3)


PROWS = ITEMS * EMBED // 128
BLK = 5000
RB = PROWS // BLK


def _prefill_body(table_ref, out_ref):
    out_ref[...] = table_ref[...]


def _tc_prefill(table128):
    return pl.pallas_call(
        _prefill_body,
        grid=(RB, B),
        in_specs=[pl.BlockSpec((BLK, 128), lambda rb, g: (rb, 0))],
        out_specs=pl.BlockSpec((BLK, 128), lambda rb, g: (g * RB + rb, 0)),
        out_shape=jax.ShapeDtypeStruct((B * PROWS, 128), jnp.float32),
    )(table128)


@functools.partial(
    pl.kernel,
    out_type=(),
    mesh=_SC_MESH,
    scratch_types=(
        pltpu.VMEM((NCH, CHUNK), jnp.int32),
        pltpu.VMEM((PER_W, EMBED), jnp.float32),
        pltpu.SemaphoreType.DMA,
    ),
    compiler_params=_SC_PARAMS,
)
def _sc_scatter(out_ref, nodes2_hbm, val_hbm, idx_v, val_v, sem):
    wid = lax.axis_index("s") * NC + lax.axis_index("c")
    base = wid * PER_W
    g = base // SEG
    pltpu.sync_copy(nodes2_hbm.at[pl.ds(wid * NCH, NCH)], idx_v)
    pltpu.sync_copy(val_hbm.at[pl.ds(base, PER_W)], val_v)
    off = g * ITEMS
    for j in range(NCH):
        for k in range(CHUNK // 16):
            sl = pl.ds(k * 16, 16)
            idx_v[j, sl] = idx_v[j, sl] + off
    copies = []
    for j in range(NCH):
        copies.append(pltpu.async_copy(
            val_v.at[pl.ds(j * CHUNK, CHUNK)], out_ref.at[idx_v.at[j]], sem))
    for c in copies:
        c.wait()


def kernel(nodes, nodes_output, ptr, table, alpha):
    del ptr
    nodes2 = nodes.reshape(NPOS // CHUNK, CHUNK)
    alpha16 = jnp.broadcast_to(alpha, (ITEMS, 16))
    trow, aval16 = _sc_gather(nodes2, table, alpha16)
    aval = aval16[:, :1]

    nodes3 = nodes.reshape(B, 1, SEG)
    feat3 = nodes_output.reshape(B, SEG, EMBED)
    trow3 = trow.reshape(B, SEG, EMBED)
    a3 = aval.reshape(B, 1, SEG)
    val = _tc_val(nodes3, feat3, trow3, a3)

    prefilled = _tc_prefill(table.reshape(PROWS, 128))
    out_ref = jax.new_ref(prefilled.reshape(B * ITEMS, EMBED))
    _sc_scatter(out_ref, nodes2, val.reshape(NPOS, EMBED))
    return out_ref[...].reshape(B, ITEMS, EMBED)


# speedup vs baseline: 1.7534x; 1.1357x over previous
import functools

import jax
import jax.numpy as jnp
from jax import lax
from jax.experimental import pallas as pl
from jax.experimental.pallas import tpu as pltpu
from jax.experimental.pallas import tpu_sc as plsc

ITEMS = 100000
EMBED = 32
B = 8
SEG = 1024
NPOS = B * SEG

NC = 2
NS = 16
NW = NC * NS
PER_W = NPOS // NW
CHUNK = 128
NCH = PER_W // CHUNK

_SC_MESH = plsc.VectorSubcoreMesh(
    core_axis_name="c", subcore_axis_name="s", num_cores=NC, num_subcores=NS)
_SC_PARAMS = pltpu.CompilerParams(
    use_tc_tiling_on_sc=False, needs_layout_passes=False)


@functools.partial(
    pl.kernel,
    out_type=(
        jax.ShapeDtypeStruct((NPOS, EMBED), jnp.float32),
        jax.ShapeDtypeStruct((NPOS // CHUNK, CHUNK), jnp.float32),
    ),
    mesh=_SC_MESH,
    scratch_types=(
        pltpu.VMEM((NCH, CHUNK), jnp.int32),
        pltpu.VMEM((PER_W, EMBED), jnp.float32),
        pltpu.VMEM((NCH, CHUNK), jnp.float32),
        pltpu.VMEM((ITEMS,), jnp.float32),
        pltpu.SemaphoreType.DMA,
    ),
    compiler_params=_SC_PARAMS,
)
def _sc_gather(nodes2_hbm, table_hbm, alpha_hbm, trow_hbm, aval_hbm,
               idx_v, rows_v, a_v, alpha_v, sem):
    wid = lax.axis_index("s") * NC + lax.axis_index("c")
    base = wid * PER_W
    pltpu.sync_copy(nodes2_hbm.at[pl.ds(wid * NCH, NCH)], idx_v)
    alpha_cp = pltpu.async_copy(alpha_hbm, alpha_v, sem)
    copies = []
    for j in range(NCH):
        copies.append(pltpu.async_copy(
            table_hbm.at[idx_v.at[j]], rows_v.at[pl.ds(j * CHUNK, CHUNK)], sem))
    alpha_cp.wait()
    for j in range(NCH):
        for k in range(CHUNK // 16):
            sl = pl.ds(k * 16, 16)
            a_v[j, sl] = plsc.load_gather(alpha_v, [idx_v[j, sl]])
    for c in copies:
        c.wait()
    pltpu.sync_copy(rows_v, trow_hbm.at[pl.ds(base, PER_W)])
    pltpu.sync_copy(a_v, aval_hbm.at[pl.ds(wid * NCH, NCH)])


def _val_body(nodes_ref, feat_ref, trow_ref, a_ref, val_ref):
    seg = nodes_ref[0, 0, :]
    eq = (seg[:, None] == seg[None, :]).astype(jnp.float32)
    counts = jnp.sum(eq, axis=1)
    sums = jnp.dot(eq, feat_ref[0],
                   preferred_element_type=jnp.float32)
    mean = sums / counts[:, None]
    a = a_ref[0, 0, :][:, None]
    val_ref[0] = (1.0 - a) * trow_ref[0] + a * mean


def _tc_val(nodes3, feat3, trow3, a3):
    return pl.pallas_call(
        _val_body,
        grid=(B,),
        in_specs=[
            pl.BlockSpec((1, 1, SEG), lambda g: (g, 0, 0)),
            pl.BlockSpec((1, SEG, EMBED), lambda g: (g, 0, 0)),
            pl.BlockSpec((1, SEG, EMBED), lambda g: (g, 0, 0)),
            pl.BlockSpec((1, 1, SEG), lambda g: (g, 0, 0)),
        ],
        out_specs=pl.BlockSpec((1, SEG, EMBED), lambda g: (g, 0, 0)),
        out_shape=jax.ShapeDtypeStruct((B, SEG, EMBED), jnp.float32),
    )(nodes3, feat3, trow3, a3)


PROWS = ITEMS * EMBED // 128
BLK = 5000
RB = PROWS // BLK


def _prefill_body(table_ref, out_ref):
    out_ref[...] = table_ref[...]


def _tc_prefill(table128):
    return pl.pallas_call(
        _prefill_body,
        grid=(RB, B),
        in_specs=[pl.BlockSpec((BLK, 128), lambda rb, g: (rb, 0))],
        out_specs=pl.BlockSpec((BLK, 128), lambda rb, g: (g * RB + rb, 0)),
        out_shape=jax.ShapeDtypeStruct((B * PROWS, 128), jnp.float32),
    )(table128)


@functools.partial(
    pl.kernel,
    out_type=(),
    mesh=_SC_MESH,
    scratch_types=(
        pltpu.VMEM((NCH, CHUNK), jnp.int32),
        pltpu.VMEM((PER_W, EMBED), jnp.float32),
        pltpu.SemaphoreType.DMA,
    ),
    compiler_params=_SC_PARAMS,
)
def _sc_scatter(out_ref, nodes2_hbm, val_hbm, idx_v, val_v, sem):
    wid = lax.axis_index("s") * NC + lax.axis_index("c")
    base = wid * PER_W
    g = base // SEG
    pltpu.sync_copy(nodes2_hbm.at[pl.ds(wid * NCH, NCH)], idx_v)
    pltpu.sync_copy(val_hbm.at[pl.ds(base, PER_W)], val_v)
    off = g * ITEMS
    for j in range(NCH):
        for k in range(CHUNK // 16):
            sl = pl.ds(k * 16, 16)
            idx_v[j, sl] = idx_v[j, sl] + off
    copies = []
    for j in range(NCH):
        copies.append(pltpu.async_copy(
            val_v.at[pl.ds(j * CHUNK, CHUNK)], out_ref.at[idx_v.at[j]], sem))
    for c in copies:
        c.wait()


def kernel(nodes, nodes_output, ptr, table, alpha):
    del ptr
    nodes2 = nodes.reshape(NPOS // CHUNK, CHUNK)
    tflat = table.reshape(ITEMS * EMBED)
    trow, avalr = _sc_gather(nodes2, tflat.reshape(ITEMS, EMBED),
                             alpha.reshape(ITEMS))
    aval = avalr.reshape(NPOS, 1)

    nodes3 = nodes.reshape(B, 1, SEG)
    feat3 = nodes_output.reshape(B, SEG, EMBED)
    trow3 = trow.reshape(B, SEG, EMBED)
    a3 = aval.reshape(B, 1, SEG)
    val = _tc_val(nodes3, feat3, trow3, a3)

    prefilled = _tc_prefill(tflat.reshape(PROWS, 128))
    out_ref = jax.new_ref(prefilled.reshape(B * ITEMS, EMBED))
    _sc_scatter(out_ref, nodes2, val.reshape(NPOS, EMBED))
    return out_ref[...].reshape(B, ITEMS, EMBED)


# speedup vs baseline: 2.3121x; 1.3186x over previous
import functools

import jax
import jax.numpy as jnp
from jax import lax
from jax.experimental import pallas as pl
from jax.experimental.pallas import tpu as pltpu
from jax.experimental.pallas import tpu_sc as plsc

ITEMS = 100000
EMBED = 32
B = 8
SEG = 1024
NPOS = B * SEG

NC = 2
NS = 16
NW = NC * NS
PER_W = NPOS // NW
CHUNK = 128
NCH = PER_W // CHUNK

_SC_MESH = plsc.VectorSubcoreMesh(
    core_axis_name="c", subcore_axis_name="s", num_cores=NC, num_subcores=NS)
_SC_PARAMS = pltpu.CompilerParams(
    use_tc_tiling_on_sc=False, needs_layout_passes=False)


@functools.partial(
    pl.kernel,
    out_type=(
        jax.ShapeDtypeStruct((NPOS, EMBED), jnp.float32),
        jax.ShapeDtypeStruct((NPOS // CHUNK, CHUNK), jnp.float32),
    ),
    mesh=_SC_MESH,
    scratch_types=(
        pltpu.VMEM((NCH, CHUNK), jnp.int32),
        pltpu.VMEM((PER_W, EMBED), jnp.float32),
        pltpu.VMEM((NCH, CHUNK), jnp.float32),
        pltpu.VMEM((ITEMS,), jnp.float32),
        pltpu.SemaphoreType.DMA,
    ),
    compiler_params=_SC_PARAMS,
)
def _sc_gather(nodes2_hbm, table_hbm, alpha_hbm, trow_hbm, aval_hbm,
               idx_v, rows_v, a_v, alpha_v, sem):
    wid = lax.axis_index("s") * NC + lax.axis_index("c")
    base = wid * PER_W
    pltpu.sync_copy(nodes2_hbm.at[pl.ds(wid * NCH, NCH)], idx_v)
    alpha_cp = pltpu.async_copy(alpha_hbm, alpha_v, sem)
    copies = []
    for j in range(NCH):
        copies.append(pltpu.async_copy(
            table_hbm.at[idx_v.at[j]], rows_v.at[pl.ds(j * CHUNK, CHUNK)], sem))
    alpha_cp.wait()
    for j in range(NCH):
        for k in range(CHUNK // 16):
            sl = pl.ds(k * 16, 16)
            a_v[j, sl] = plsc.load_gather(alpha_v, [idx_v[j, sl]])
    for c in copies:
        c.wait()
    pltpu.sync_copy(rows_v, trow_hbm.at[pl.ds(base, PER_W)])
    pltpu.sync_copy(a_v, aval_hbm.at[pl.ds(wid * NCH, NCH)])


def _val_body(nodes_ref, feat_ref, trow_ref, a_ref, val_ref):
    seg = nodes_ref[0, 0, :]
    eq = (seg[:, None] == seg[None, :]).astype(jnp.float32)
    counts = jnp.sum(eq, axis=1)
    sums = jnp.dot(eq, feat_ref[0],
                   preferred_element_type=jnp.float32)
    mean = sums / counts[:, None]
    a = a_ref[0, 0, :][:, None]
    val_ref[0] = (1.0 - a) * trow_ref[0] + a * mean


def _tc_val(nodes3, feat3, trow3, a3):
    return pl.pallas_call(
        _val_body,
        grid=(B,),
        in_specs=[
            pl.BlockSpec((1, 1, SEG), lambda g: (g, 0, 0)),
            pl.BlockSpec((1, SEG, EMBED), lambda g: (g, 0, 0)),
            pl.BlockSpec((1, SEG, EMBED), lambda g: (g, 0, 0)),
            pl.BlockSpec((1, 1, SEG), lambda g: (g, 0, 0)),
        ],
        out_specs=pl.BlockSpec((1, SEG, EMBED), lambda g: (g, 0, 0)),
        out_shape=jax.ShapeDtypeStruct((B, SEG, EMBED), jnp.float32),
    )(nodes3, feat3, trow3, a3)


PROWS = ITEMS * EMBED // 128
BLK = 5000
RB = PROWS // BLK


def _prefill_body(table_ref, out_ref):
    out_ref[...] = table_ref[...]


def _tc_prefill(table128):
    return pl.pallas_call(
        _prefill_body,
        grid=(RB, B),
        in_specs=[pl.BlockSpec((BLK, 128), lambda rb, g: (rb, 0))],
        out_specs=pl.BlockSpec((BLK, 128), lambda rb, g: (g * RB + rb, 0)),
        out_shape=jax.ShapeDtypeStruct((B * PROWS, 128), jnp.float32),
    )(table128)


@functools.partial(
    pl.kernel,
    out_type=(),
    mesh=_SC_MESH,
    scratch_types=(
        pltpu.VMEM((NCH, CHUNK), jnp.int32),
        pltpu.VMEM((PER_W, EMBED), jnp.float32),
        pltpu.SemaphoreType.DMA,
    ),
    compiler_params=_SC_PARAMS,
)
def _sc_scatter(out_ref, nodes2_hbm, val_hbm, idx_v, val_v, sem):
    wid = lax.axis_index("s") * NC + lax.axis_index("c")
    base = wid * PER_W
    g = base // SEG
    pltpu.sync_copy(nodes2_hbm.at[pl.ds(wid * NCH, NCH)], idx_v)
    pltpu.sync_copy(val_hbm.at[pl.ds(base, PER_W)], val_v)
    off = g * ITEMS
    for j in range(NCH):
        for k in range(CHUNK // 16):
            sl = pl.ds(k * 16, 16)
            idx_v[j, sl] = idx_v[j, sl] + off
    copies = []
    for j in range(NCH):
        copies.append(pltpu.async_copy(
            val_v.at[pl.ds(j * CHUNK, CHUNK)], out_ref.at[idx_v.at[j]], sem))
    for c in copies:
        c.wait()


def kernel(nodes, nodes_output, ptr, table, alpha):
    del ptr
    nodes2 = nodes.reshape(NPOS // CHUNK, CHUNK)
    tflat = table.reshape(ITEMS * EMBED)
    trow, avalr = _sc_gather(nodes2, tflat.reshape(ITEMS, EMBED),
                             alpha.reshape(ITEMS))
    aval = avalr.reshape(NPOS, 1)

    nodes3 = nodes.reshape(B, 1, SEG)
    feat3 = nodes_output.reshape(B, SEG, EMBED)
    trow3 = trow.reshape(B, SEG, EMBED)
    a3 = aval.reshape(B, 1, SEG)
    val = _tc_val(nodes3, feat3, trow3, a3)

    prefilled = _tc_prefill(tflat.reshape(PROWS, 128))
    out_ref = jax.new_ref(prefilled.reshape(B * ITEMS, EMBED))
    _sc_scatter(out_ref, nodes2, val.reshape(NPOS, EMBED))
    out = out_ref[...].reshape(B, ITEMS, EMBED)
    from jax.experimental import layout as _jl
    return _jl.with_layout_constraint(
        out, _jl.Layout(major_to_minor=(0, 1, 2), tiling=((1024,),)))
